# baseline (speedup 1.0000x reference)
import functools

import jax
import jax.numpy as jnp
from jax import lax
from jax.experimental import pallas as pl
from jax.experimental.pallas import tpu as pltpu
from jax.experimental.pallas import tpu_sc as plsc

_NUM_CORES = 2
_NUM_SUBCORES = 16
_NW = _NUM_CORES * _NUM_SUBCORES

_CHUNK = 128
_NBUF = 8


@functools.lru_cache(maxsize=None)
def _make_sc_gather(n_rows: int, d: int):
  assert n_rows % (_NW * _CHUNK) == 0, n_rows
  nch = n_rows // (_NW * _CHUNK)
  nbuf = _NBUF
  lag = nbuf // 2
  assert nbuf % 2 == 0 and nch % nbuf == 0 and nch >= 2 * nbuf
  g_total = (nch - 2 * lag) // nbuf

  mesh = plsc.VectorSubcoreMesh(core_axis_name="c", subcore_axis_name="s")

  @functools.partial(
      pl.kernel,
      mesh=mesh,
      out_type=jax.ShapeDtypeStruct((_NW * nch, _CHUNK, d), jnp.float32),
      compiler_params=pltpu.CompilerParams(use_tc_tiling_on_sc=False),
      scratch_types=(
          [pltpu.VMEM((nch, _CHUNK), jnp.int32)]
          + [pltpu.VMEM((_CHUNK, d), jnp.float32) for _ in range(nbuf)]
          + [pltpu.SemaphoreType.DMA for _ in range(2 * nbuf)]
      ),
  )
  def sc_gather(idx_hbm, table_hbm, out_hbm, idx_v, *rest):
    rbufs = rest[:nbuf]
    gsems = rest[nbuf:2 * nbuf]
    wsems = rest[2 * nbuf:]
    wid = lax.axis_index("s") * _NUM_CORES + lax.axis_index("c")

    pltpu.sync_copy(idx_hbm.at[wid], idx_v)

    def start_gather(b, j):
      pltpu.async_copy(table_hbm.at[idx_v.at[j]], rbufs[b], gsems[b])

    def wait_gather(b, j):
      pltpu.make_async_copy(
          table_hbm.at[idx_v.at[j]], rbufs[b], gsems[b]).wait()

    def start_write(b, j):
      pltpu.async_copy(rbufs[b], out_hbm.at[wid * nch + j], wsems[b])

    def wait_write(b, j):
      pltpu.make_async_copy(
          rbufs[b], out_hbm.at[wid * nch + j], wsems[b]).wait()

    for b in range(lag):
      start_gather(b, b)
    for t in range(lag):
      wait_gather(t, t)
      start_write(t, t)
      start_gather(t + lag, t + lag)

    def body(g, carry):
      t0 = lag + g * nbuf
      for b in range(nbuf):
        t = t0 + b
        rb = (lag + b) % nbuf
        wait_gather(rb, t)
        start_write(rb, t)
        pb = b % nbuf
        wait_write(pb, t + lag - nbuf)
        start_gather(pb, t + lag)
      return carry

    lax.fori_loop(0, g_total, body, 0)

    for b in range(lag):
      t = nch - lag + b
      wait_gather((lag + b) % nbuf, t)
      start_write((lag + b) % nbuf, t)
    for i in range(nbuf):
      wait_write(i, nch - nbuf + i)

  return sc_gather


def kernel(indices, table):
  b, s = indices.shape
  v, d = table.shape
  n = b * s
  idx3 = indices.astype(jnp.int32).reshape(_NW, n // (_NW * _CHUNK), _CHUNK)
  out = _make_sc_gather(n, d)(idx3, table)
  return out.reshape(b, s, d)
